# Initial kernel scaffold; baseline (speedup 1.0000x reference)
import jax
import jax.numpy as jnp
from jax.experimental import pallas as pl


def kernel(position, div_term):
    raise NotImplementedError("write your pallas kernel here")


# speedup vs baseline: 6.4777x; 6.4777x over previous
import functools
import math

import jax
import jax.numpy as jnp
from jax import lax
from jax.experimental import pallas as pl
from jax.experimental.pallas import tpu as pltpu
from jax.experimental.pallas import tpu_sc as plsc

D_MODEL = 128
MAX_LEN = 2048

_NUM_CORES = 2
_NUM_SUBCORES = 16
_NUM_WORKERS = _NUM_CORES * _NUM_SUBCORES


def _table_body(div_full_ref, out_ref):
    pos = lax.broadcasted_iota(jnp.int32, (MAX_LEN, D_MODEL), 0).astype(jnp.float32)
    angles = pos * div_full_ref[...]
    lane = lax.broadcasted_iota(jnp.int32, (MAX_LEN, D_MODEL), 1)
    out_ref[...] = jnp.where(lane % 2 == 0, jnp.sin(angles), jnp.cos(angles))


def _build_table(div_term):
    div_full = jnp.repeat(div_term, 2).reshape(1, D_MODEL)
    return pl.pallas_call(
        _table_body,
        out_shape=jax.ShapeDtypeStruct((MAX_LEN, D_MODEL), jnp.float32),
    )(div_full)


def _make_gather(batch, chunk):
    b_per_w = batch // _NUM_WORKERS
    n_chunks = b_per_w // chunk
    mesh = plsc.VectorSubcoreMesh(core_axis_name="c", subcore_axis_name="s")

    @functools.partial(
        pl.kernel,
        mesh=mesh,
        out_type=jax.ShapeDtypeStruct((batch, D_MODEL), jnp.float32),
        scratch_types=[
            pltpu.VMEM((chunk,), jnp.int32),
            pltpu.VMEM((chunk, D_MODEL), jnp.float32),
            pltpu.SemaphoreType.DMA,
        ],
    )
    def gather(table_hbm, idx_hbm, out_hbm, idx_v, rows_v, sem):
        wid = lax.axis_index("s") * _NUM_CORES + lax.axis_index("c")
        base = wid * b_per_w

        def body(j, carry):
            off = base + j * chunk
            pltpu.sync_copy(idx_hbm.at[pl.ds(off, chunk)], idx_v)
            pltpu.async_copy(table_hbm.at[idx_v], rows_v, sem).wait()
            pltpu.sync_copy(rows_v, out_hbm.at[pl.ds(off, chunk)])
            return carry

        lax.fori_loop(0, n_chunks, body, 0)

    return gather


def kernel(position, div_term):
    table = _build_table(div_term)
    idx = position.reshape(-1)
    batch = idx.shape[0]
    gather = _make_gather(batch, chunk=256)
    return gather(table, idx)


# speedup vs baseline: 8.0367x; 1.2407x over previous
import functools
import math

import jax
import jax.numpy as jnp
from jax import lax
from jax.experimental import pallas as pl
from jax.experimental.pallas import tpu as pltpu
from jax.experimental.pallas import tpu_sc as plsc

D_MODEL = 128
MAX_LEN = 2048

_NUM_CORES = 2
_NUM_SUBCORES = 16
_NUM_WORKERS = _NUM_CORES * _NUM_SUBCORES


def _table_body(div_full_ref, out_ref):
    pos = lax.broadcasted_iota(jnp.int32, (MAX_LEN, D_MODEL), 0).astype(jnp.float32)
    angles = pos * div_full_ref[...]
    lane = lax.broadcasted_iota(jnp.int32, (MAX_LEN, D_MODEL), 1)
    out_ref[...] = jnp.where(lane % 2 == 0, jnp.sin(angles), jnp.cos(angles))


def _build_table(div_term):
    div_full = jnp.repeat(div_term, 2).reshape(1, D_MODEL)
    return pl.pallas_call(
        _table_body,
        out_shape=jax.ShapeDtypeStruct((MAX_LEN, D_MODEL), jnp.float32),
    )(div_full)


def _make_gather(batch, chunk):
    b_per_w = batch // _NUM_WORKERS
    n_chunks = b_per_w // chunk
    assert n_chunks % 2 == 0 and n_chunks >= 4
    mesh = plsc.VectorSubcoreMesh(core_axis_name="c", subcore_axis_name="s")

    @functools.partial(
        pl.kernel,
        mesh=mesh,
        out_type=jax.ShapeDtypeStruct((batch, D_MODEL), jnp.float32),
        scratch_types=[
            pltpu.VMEM((b_per_w,), jnp.int32),
            pltpu.VMEM((2, chunk, D_MODEL), jnp.float32),
            pltpu.SemaphoreType.DMA,
            pltpu.SemaphoreType.DMA,
            pltpu.SemaphoreType.DMA,
            pltpu.SemaphoreType.DMA,
        ],
    )
    def gather(table_hbm, idx_hbm, out_hbm, idx_v, rows_v, gs0, gs1, os0, os1):
        wid = lax.axis_index("s") * _NUM_CORES + lax.axis_index("c")
        base = wid * b_per_w
        pltpu.sync_copy(idx_hbm.at[pl.ds(base, b_per_w)], idx_v)
        gsems = (gs0, gs1)
        osems = (os0, os1)

        def gather_desc(j, b):
            return pltpu.make_async_copy(
                table_hbm.at[idx_v.at[pl.ds(j * chunk, chunk)]],
                rows_v.at[b], gsems[b])

        def out_desc(j, b):
            return pltpu.make_async_copy(
                rows_v.at[b], out_hbm.at[pl.ds(base + j * chunk, chunk)],
                osems[b])

        gather_desc(0, 0).start()
        gather_desc(1, 1).start()

        def body(j2, carry):
            for b in range(2):
                j = j2 * 2 + b
                gather_desc(j, b).wait()
                out_desc(j, b).start()
                out_desc(j, b).wait()

                @pl.when(j + 2 < n_chunks)
                def _():
                    gather_desc(j + 2, b).start()

            return carry

        lax.fori_loop(0, n_chunks // 2, body, 0)

    return gather


def kernel(position, div_term):
    table = _build_table(div_term)
    idx = position.reshape(-1)
    batch = idx.shape[0]
    gather = _make_gather(batch, chunk=320)
    return gather(table, idx)


# speedup vs baseline: 15.6827x; 1.9514x over previous
import functools
import math

import jax
import jax.numpy as jnp
from jax import lax
from jax.experimental import pallas as pl
from jax.experimental.pallas import tpu as pltpu
from jax.experimental.pallas import tpu_sc as plsc

D_MODEL = 128
MAX_LEN = 2048

_NUM_CORES = 2
_NUM_SUBCORES = 16
_NUM_WORKERS = _NUM_CORES * _NUM_SUBCORES


def _table_body(div_full_ref, out_ref):
    pos = lax.broadcasted_iota(jnp.int32, (MAX_LEN, D_MODEL), 0).astype(jnp.float32)
    angles = pos * div_full_ref[...]
    lane = lax.broadcasted_iota(jnp.int32, (MAX_LEN, D_MODEL), 1)
    out_ref[...] = jnp.where(lane % 2 == 0, jnp.sin(angles), jnp.cos(angles))


def _build_table(div_term):
    div_full = jnp.repeat(div_term, 2).reshape(1, D_MODEL)
    return pl.pallas_call(
        _table_body,
        out_shape=jax.ShapeDtypeStruct((MAX_LEN, D_MODEL), jnp.float32),
    )(div_full)


def _make_gather(batch, chunk):
    b_per_w = batch // _NUM_WORKERS
    n_chunks = b_per_w // chunk
    assert n_chunks % 2 == 0 and n_chunks >= 4
    mesh = plsc.VectorSubcoreMesh(core_axis_name="c", subcore_axis_name="s")

    @functools.partial(
        pl.kernel,
        mesh=mesh,
        out_type=jax.ShapeDtypeStruct((batch, D_MODEL), jnp.float32),
        scratch_types=[
            pltpu.VMEM((b_per_w,), jnp.int32),
            pltpu.VMEM((2, chunk, D_MODEL), jnp.float32),
            pltpu.VMEM_SHARED((MAX_LEN, D_MODEL), jnp.float32),
            pltpu.SemaphoreType.DMA,
            pltpu.SemaphoreType.DMA,
            pltpu.SemaphoreType.DMA,
            pltpu.SemaphoreType.DMA,
        ],
    )
    def gather(table_hbm, idx_hbm, out_hbm, idx_v, rows_v, table_sp,
               gs0, gs1, os0, os1):
        wid = lax.axis_index("s") * _NUM_CORES + lax.axis_index("c")
        base = wid * b_per_w

        @pl.when(lax.axis_index("s") == 0)
        def _():
            pltpu.sync_copy(table_hbm, table_sp)

        pltpu.sync_copy(idx_hbm.at[pl.ds(base, b_per_w)], idx_v)
        plsc.subcore_barrier()
        gsems = (gs0, gs1)
        osems = (os0, os1)

        def gather_desc(j, b):
            return pltpu.make_async_copy(
                table_sp.at[idx_v.at[pl.ds(j * chunk, chunk)]],
                rows_v.at[b], gsems[b])

        def out_desc(j, b):
            return pltpu.make_async_copy(
                rows_v.at[b], out_hbm.at[pl.ds(base + j * chunk, chunk)],
                osems[b])

        gather_desc(0, 0).start()
        gather_desc(1, 1).start()

        def body(j2, carry):
            for b in range(2):
                j = j2 * 2 + b
                gather_desc(j, b).wait()
                out_desc(j, b).start()
                out_desc(j, b).wait()

                @pl.when(j + 2 < n_chunks)
                def _():
                    gather_desc(j + 2, b).start()

            return carry

        lax.fori_loop(0, n_chunks // 2, body, 0)

    return gather


def kernel(position, div_term):
    table = _build_table(div_term)
    idx = position.reshape(-1)
    batch = idx.shape[0]
    gather = _make_gather(batch, chunk=320)
    return gather(table, idx)


# speedup vs baseline: 16.0843x; 1.0256x over previous
import functools
import math

import jax
import jax.numpy as jnp
from jax import lax
from jax.experimental import pallas as pl
from jax.experimental.pallas import tpu as pltpu
from jax.experimental.pallas import tpu_sc as plsc

D_MODEL = 128
MAX_LEN = 2048

_NUM_CORES = 2
_NUM_SUBCORES = 16
_NUM_WORKERS = _NUM_CORES * _NUM_SUBCORES


def _table_body(div_full_ref, out_ref):
    pos = lax.broadcasted_iota(jnp.int32, (MAX_LEN, D_MODEL), 0).astype(jnp.float32)
    angles = pos * div_full_ref[...]
    lane = lax.broadcasted_iota(jnp.int32, (MAX_LEN, D_MODEL), 1)
    out_ref[...] = jnp.where(lane % 2 == 0, jnp.sin(angles), jnp.cos(angles))


def _build_table(div_term):
    div_full = jnp.repeat(div_term, 2).reshape(1, D_MODEL)
    return pl.pallas_call(
        _table_body,
        out_shape=jax.ShapeDtypeStruct((MAX_LEN, D_MODEL), jnp.float32),
    )(div_full)


_NBUF = 4
_LOOKAHEAD = 2


def _make_gather(batch, chunk):
    b_per_w = batch // _NUM_WORKERS
    n_chunks = b_per_w // chunk
    assert n_chunks % _NBUF == 0 and n_chunks >= 2 * _NBUF
    mesh = plsc.VectorSubcoreMesh(core_axis_name="c", subcore_axis_name="s")

    @functools.partial(
        pl.kernel,
        mesh=mesh,
        out_type=jax.ShapeDtypeStruct((batch, D_MODEL), jnp.float32),
        scratch_types=[
            pltpu.VMEM((b_per_w,), jnp.int32),
            pltpu.VMEM((_NBUF, chunk, D_MODEL), jnp.float32),
            pltpu.VMEM_SHARED((MAX_LEN, D_MODEL), jnp.float32),
            [pltpu.SemaphoreType.DMA] * _NBUF,
            [pltpu.SemaphoreType.DMA] * _NBUF,
        ],
    )
    def gather(table_hbm, idx_hbm, out_hbm, idx_v, rows_v, table_sp,
               gsems, osems):
        wid = lax.axis_index("s") * _NUM_CORES + lax.axis_index("c")
        base = wid * b_per_w

        @pl.when(lax.axis_index("s") == 0)
        def _():
            pltpu.sync_copy(table_hbm, table_sp)

        pltpu.sync_copy(idx_hbm.at[pl.ds(base, b_per_w)], idx_v)
        plsc.subcore_barrier()

        def gather_desc(j, b):
            return pltpu.make_async_copy(
                table_sp.at[idx_v.at[pl.ds(j * chunk, chunk)]],
                rows_v.at[b], gsems[b])

        def out_desc(j, b):
            return pltpu.make_async_copy(
                rows_v.at[b], out_hbm.at[pl.ds(base + j * chunk, chunk)],
                osems[b])

        for j in range(_LOOKAHEAD):
            gather_desc(j, j % _NBUF).start()

        def step(j, jd, b, bd):
            @pl.when(jd >= _NBUF)
            def _():
                out_desc(jd - _NBUF, bd).wait()

            @pl.when(jd < n_chunks)
            def _():
                gather_desc(jd, bd).start()

            gather_desc(j, b).wait()
            out_desc(j, b).start()

        def body(j2, carry):
            for u in range(_NBUF):
                j = j2 * _NBUF + u
                jd = j + _LOOKAHEAD
                step(j, jd, u, (u + _LOOKAHEAD) % _NBUF)
            return carry

        lax.fori_loop(0, n_chunks // _NBUF, body, 0)

        for j in range(n_chunks - _NBUF + _LOOKAHEAD, n_chunks):
            out_desc(j, j % _NBUF).wait()

    return gather


def kernel(position, div_term):
    table = _build_table(div_term)
    idx = position.reshape(-1)
    batch = idx.shape[0]
    gather = _make_gather(batch, chunk=160)
    return gather(table, idx)


# speedup vs baseline: 16.1245x; 1.0025x over previous
import functools
import math

import jax
import jax.numpy as jnp
from jax import lax
from jax.experimental import pallas as pl
from jax.experimental.pallas import tpu as pltpu
from jax.experimental.pallas import tpu_sc as plsc

D_MODEL = 128
MAX_LEN = 2048

_NUM_CORES = 2
_NUM_SUBCORES = 16
_NUM_WORKERS = _NUM_CORES * _NUM_SUBCORES


def _table_body(div_full_ref, out_ref):
    pos = lax.broadcasted_iota(jnp.int32, (MAX_LEN, D_MODEL), 0).astype(jnp.float32)
    angles = pos * div_full_ref[...]
    lane = lax.broadcasted_iota(jnp.int32, (MAX_LEN, D_MODEL), 1)
    out_ref[...] = jnp.where(lane % 2 == 0, jnp.sin(angles), jnp.cos(angles))


def _build_table(div_term):
    div_full = jnp.repeat(div_term, 2).reshape(1, D_MODEL)
    return pl.pallas_call(
        _table_body,
        out_shape=jax.ShapeDtypeStruct((MAX_LEN, D_MODEL), jnp.float32),
    )(div_full)


_NBUF = 5
_LOOKAHEAD = 2


def _make_gather(batch, chunk):
    b_per_w = batch // _NUM_WORKERS
    n_chunks = b_per_w // chunk
    assert n_chunks % _NBUF == 0 and n_chunks >= 2 * _NBUF
    mesh = plsc.VectorSubcoreMesh(core_axis_name="c", subcore_axis_name="s")

    @functools.partial(
        pl.kernel,
        mesh=mesh,
        out_type=jax.ShapeDtypeStruct((batch, D_MODEL), jnp.float32),
        scratch_types=[
            pltpu.VMEM((b_per_w,), jnp.int32),
            pltpu.VMEM((_NBUF, chunk, D_MODEL), jnp.float32),
            pltpu.VMEM_SHARED((MAX_LEN, D_MODEL), jnp.float32),
            [pltpu.SemaphoreType.DMA] * _NBUF,
            [pltpu.SemaphoreType.DMA] * _NBUF,
        ],
    )
    def gather(table_hbm, idx_hbm, out_hbm, idx_v, rows_v, table_sp,
               gsems, osems):
        wid = lax.axis_index("s") * _NUM_CORES + lax.axis_index("c")
        base = wid * b_per_w

        @pl.when(lax.axis_index("s") == 0)
        def _():
            pltpu.sync_copy(table_hbm, table_sp)

        pltpu.sync_copy(idx_hbm.at[pl.ds(base, b_per_w)], idx_v)
        plsc.subcore_barrier()

        def gather_desc(j, b):
            return pltpu.make_async_copy(
                table_sp.at[idx_v.at[pl.ds(j * chunk, chunk)]],
                rows_v.at[b], gsems[b])

        def out_desc(j, b):
            return pltpu.make_async_copy(
                rows_v.at[b], out_hbm.at[pl.ds(base + j * chunk, chunk)],
                osems[b])

        for j in range(_LOOKAHEAD):
            gather_desc(j, j % _NBUF).start()

        def step(j, jd, b, bd):
            @pl.when(jd >= _NBUF)
            def _():
                out_desc(jd - _NBUF, bd).wait()

            @pl.when(jd < n_chunks)
            def _():
                gather_desc(jd, bd).start()

            gather_desc(j, b).wait()
            out_desc(j, b).start()

        def body(j2, carry):
            for u in range(_NBUF):
                j = j2 * _NBUF + u
                jd = j + _LOOKAHEAD
                step(j, jd, u, (u + _LOOKAHEAD) % _NBUF)
            return carry

        lax.fori_loop(0, n_chunks // _NBUF, body, 0)

        for j in range(n_chunks - _NBUF + _LOOKAHEAD, n_chunks):
            out_desc(j, j % _NBUF).wait()

    return gather


def kernel(position, div_term):
    table = _build_table(div_term)
    idx = position.reshape(-1)
    batch = idx.shape[0]
    gather = _make_gather(batch, chunk=128)
    return gather(table, idx)
